# Initial kernel scaffold; baseline (speedup 1.0000x reference)
import jax
import jax.numpy as jnp
from jax.experimental import pallas as pl


def kernel(particle_pos, log_weights, obs_dist, obs_angle, ray_map, sensor_table, u):
    raise NotImplementedError("write your pallas kernel here")


# speedup vs baseline: 32.5476x; 32.5476x over previous
import functools

import numpy as np
import jax
import jax.numpy as jnp
from jax import lax
from jax.experimental import pallas as pl
from jax.experimental.pallas import tpu as pltpu
from jax.experimental.pallas import tpu_sc as plsc

_N = 16384
_B = 360
_GRID = 256
_ABINS = 64
_DBINS = 64
_MAXD = 100.0
_EVID = 1.0
_TWO_PI = 2.0 * np.pi

_NC = 2
_NS = 16
_NW = _NC * _NS
_PPW = _N // _NW
_CHUNK = 128
_NCHUNK = _PPW // _CHUNK
_GSTARTS = tuple(list(range(0, _B - 16, 16)) + [_B - 16])

_C_ABIN = np.float32(np.float64(_ABINS) / (2.0 * np.pi))
_C_DBIN = np.float32(np.float64(_DBINS) / np.float64(_MAXD))


def _candidate_grid(centers64, width):
    c0 = centers64.astype(np.float32)
    cands = [c0]
    lo = c0.copy()
    hi = c0.copy()
    for _ in range(width):
        lo = np.nextafter(lo, np.float32(-np.inf))
        hi = np.nextafter(hi, np.float32(np.inf))
        cands.append(lo)
        cands.append(hi)
    return np.stack(cands, axis=1)


_ABIN_CANDS = _candidate_grid(np.arange(129, dtype=np.float64) * (2.0 * np.pi / 64.0), 40)
_DBIN_CANDS = _candidate_grid(np.arange(65, dtype=np.float64) * (100.0 / 64.0), 40)


def _build_thresholds(guard):
    two_pi = _TWO_PI
    s = jnp.asarray(_ABIN_CANDS) + guard
    ang = s % two_pi
    fu = jnp.clip(jnp.floor(ang / two_pi * _ABINS).astype(jnp.int32), 0, _ABINS - 1)
    fu = fu + _ABINS * (s >= np.float32(two_pi)).astype(jnp.int32)
    kk = jnp.arange(129, dtype=jnp.int32)[:, None]
    th1 = jnp.min(jnp.where(fu >= kk, s, jnp.inf), axis=1)
    th1 = th1.at[0].set(-jnp.inf)
    th1 = jnp.concatenate([th1, jnp.full((15,), jnp.inf, jnp.float32)])
    d = jnp.asarray(_DBIN_CANDS) + guard
    fu2 = jnp.clip(jnp.floor(d / _MAXD * _DBINS).astype(jnp.int32), 0, _DBINS - 1)
    kk2 = jnp.arange(65, dtype=jnp.int32)[:, None]
    th2 = jnp.min(jnp.where(fu2 >= kk2, d, jnp.inf), axis=1)
    th2 = th2.at[0].set(-jnp.inf)
    th2 = jnp.concatenate([th2, jnp.full((15,), jnp.inf, jnp.float32)])
    return th1, th2


def _bin16(x, scale, theta_ref):
    g = (x * scale).astype(jnp.int32)
    t0 = plsc.load_gather(theta_ref, [g])
    t1 = plsc.load_gather(theta_ref, [g + 1])
    return (g - 1) + (x >= t0).astype(jnp.int32) + (x >= t1).astype(jnp.int32)


def _sc_logp_kernel(pos_hbm, obs_hbm, obin_hbm, st_hbm, th1_hbm, th2_hbm,
                    ray_hbm, logp_hbm,
                    pos_v, cell_v, rays_v, obs_v, obin_v, st_v, th1_v, th2_v,
                    logp_v, sem):
    wid = lax.axis_index("s") * _NC + lax.axis_index("c")
    base = wid * _PPW
    pltpu.sync_copy(pos_hbm.at[pl.ds(base * 3, _PPW * 3)], pos_v)
    pltpu.sync_copy(obs_hbm, obs_v)
    pltpu.sync_copy(obin_hbm, obin_v)
    pltpu.sync_copy(st_hbm, st_v)
    pltpu.sync_copy(th1_hbm, th1_v)
    pltpu.sync_copy(th2_hbm, th2_v)

    iota = lax.iota(jnp.int32, 16)

    def cell_body(g, carry):
        px = (g * 16 + iota) * 3
        x = plsc.load_gather(pos_v, [px])
        y = plsc.load_gather(pos_v, [px + 1])
        ci = jnp.clip(x.astype(jnp.int32), 0, _GRID - 1)
        cj = jnp.clip(y.astype(jnp.int32), 0, _GRID - 1)
        cell_v[g // 8, pl.ds((g % 8) * 16, 16)] = ci * _GRID + cj
        return carry
    lax.fori_loop(0, _PPW // 16, cell_body, 0)

    copies = [
        pltpu.async_copy(ray_hbm.at[cell_v.at[c]],
                         rays_v.at[pl.ds(c * 128, 128)], sem)
        for c in range(_PPW // 128)
    ]
    for c in copies:
        c.wait()

    def bin_body(r, carry):
        for j in range(_ABINS // 16):
            dv = rays_v[r, pl.ds(j * 16, 16)]
            e = _bin16(dv, _C_DBIN, th2_v)
            rays_v[r, pl.ds(j * 16, 16)] = plsc.bitcast(e * _ABINS, jnp.float32)
        return carry
    lax.fori_loop(0, _PPW, bin_body, 0)

    for chunk in range(_NCHUNK):
        def p_body(p, carry):
            pg = chunk * _CHUNK + p
            th = plsc.load_gather(pos_v, [jnp.full((16,), pg * 3 + 2, jnp.int32)])
            psplat = jnp.full((16,), pg, jnp.int32)
            for goff in _GSTARTS:
                oa = obs_v[pl.ds(goff, 16)]
                s = th + oa
                f = _bin16(s, _C_ABIN, th1_v)
                ai = jnp.bitwise_and(f, _ABINS - 1)
                e64 = plsc.bitcast(plsc.load_gather(rays_v, [psplat, ai]), jnp.int32)
                sidx = e64 + obin_v[pl.ds(goff, 16)]
                logp_v[p, pl.ds(goff, 16)] = plsc.load_gather(st_v, [sidx])
            return carry
        lax.fori_loop(0, _CHUNK, p_body, 0)
        pltpu.sync_copy(
            logp_v, logp_hbm.at[pl.ds(base + chunk * _CHUNK, _CHUNK)])


@functools.cache
def _sc_logp():
    return pl.kernel(
        _sc_logp_kernel,
        out_type=jax.ShapeDtypeStruct((_N, _B), jnp.float32),
        mesh=plsc.VectorSubcoreMesh(core_axis_name="c", subcore_axis_name="s",
                                    num_cores=_NC, num_subcores=_NS),
        compiler_params=pltpu.CompilerParams(needs_layout_passes=False,
                                             use_tc_tiling_on_sc=False),
        scratch_types=[
            pltpu.VMEM((_PPW * 3,), jnp.float32),
            pltpu.VMEM((_PPW // 128, 128), jnp.int32),
            pltpu.VMEM((_PPW, _ABINS), jnp.float32),
            pltpu.VMEM((_B + 8,), jnp.float32),
            pltpu.VMEM((_B + 8,), jnp.int32),
            pltpu.VMEM((_DBINS * _DBINS,), jnp.float32),
            pltpu.VMEM((144,), jnp.float32),
            pltpu.VMEM((80,), jnp.float32),
            pltpu.VMEM((_CHUNK, _B), jnp.float32),
            pltpu.SemaphoreType.DMA,
        ],
    )


def kernel(particle_pos, log_weights, obs_dist, obs_angle, ray_map,
           sensor_table, u):
    two_pi = _TWO_PI
    obin = jnp.clip(jnp.floor(obs_dist / _MAXD * _DBINS).astype(jnp.int32),
                    0, _DBINS - 1)
    guard = jnp.where(u[0] < 2.0, jnp.float32(0.0), jnp.float32(1.0))
    th1, th2 = _build_thresholds(guard)

    pos_flat = particle_pos.reshape(-1)
    obs_pad = jnp.concatenate([obs_angle, jnp.zeros((8,), jnp.float32)])
    obin_pad = jnp.concatenate([obin, jnp.zeros((8,), jnp.int32)])
    st_flat = sensor_table.reshape(-1)

    logp = _sc_logp()(pos_flat, obs_pad, obin_pad, st_flat, th1, th2, ray_map)

    loglik = jnp.sum(logp, axis=1)
    logits = log_weights + _EVID * loglik
    logw = jax.nn.log_softmax(logits)
    w = jnp.exp(logw)
    cdf = jnp.cumsum(w)
    pos_u = (jnp.arange(_N, dtype=jnp.float32) + u[0]) / _N
    idx = jnp.clip(jnp.searchsorted(cdf, pos_u), 0, _N - 1)
    new_particles = jnp.take(particle_pos, idx, axis=0)
    mean_pose = jnp.sum(w[:, None] * particle_pos, axis=0)
    return new_particles, logits, mean_pose


# speedup vs baseline: 154.1801x; 4.7371x over previous
import functools

import numpy as np
import jax
import jax.numpy as jnp
from jax import lax
from jax.experimental import pallas as pl
from jax.experimental.pallas import tpu as pltpu
from jax.experimental.pallas import tpu_sc as plsc

_N = 16384
_B = 360
_GRID = 256
_ABINS = 64
_DBINS = 64
_MAXD = 100.0
_EVID = 1.0
_TWO_PI = 2.0 * np.pi

_NC = 2
_NS = 16
_NW = _NC * _NS
_PPW = _N // _NW
_CHUNK = 128
_NCHUNK = _PPW // _CHUNK
_GSTARTS = tuple(list(range(0, _B - 16, 16)) + [_B - 16])

_C_ABIN = np.float32(np.float64(_ABINS) / (2.0 * np.pi))
_C_DBIN = np.float32(np.float64(_DBINS) / np.float64(_MAXD))


def _candidate_grid(centers64, width):
    c0 = centers64.astype(np.float32)
    cands = [c0]
    lo = c0.copy()
    hi = c0.copy()
    for _ in range(width):
        lo = np.nextafter(lo, np.float32(-np.inf))
        hi = np.nextafter(hi, np.float32(np.inf))
        cands.append(lo)
        cands.append(hi)
    return np.stack(cands, axis=1)


_ABIN_CANDS = _candidate_grid(np.arange(129, dtype=np.float64) * (2.0 * np.pi / 64.0), 40)
_DBIN_CANDS = _candidate_grid(np.arange(65, dtype=np.float64) * (100.0 / 64.0), 40)


def _build_thresholds(guard):
    two_pi = _TWO_PI
    s = jnp.asarray(_ABIN_CANDS) + guard
    ang = s % two_pi
    fu = jnp.clip(jnp.floor(ang / two_pi * _ABINS).astype(jnp.int32), 0, _ABINS - 1)
    fu = fu + _ABINS * (s >= np.float32(two_pi)).astype(jnp.int32)
    kk = jnp.arange(129, dtype=jnp.int32)[:, None]
    th1 = jnp.min(jnp.where(fu >= kk, s, jnp.inf), axis=1)
    th1 = th1.at[0].set(-jnp.inf)
    th1 = jnp.concatenate([th1, jnp.full((15,), jnp.inf, jnp.float32)])
    d = jnp.asarray(_DBIN_CANDS) + guard
    fu2 = jnp.clip(jnp.floor(d / _MAXD * _DBINS).astype(jnp.int32), 0, _DBINS - 1)
    kk2 = jnp.arange(65, dtype=jnp.int32)[:, None]
    th2 = jnp.min(jnp.where(fu2 >= kk2, d, jnp.inf), axis=1)
    th2 = th2.at[0].set(-jnp.inf)
    th2 = jnp.concatenate([th2, jnp.full((15,), jnp.inf, jnp.float32)])
    return th1, th2


def _bin16(x, scale, theta_ref):
    g = (x * scale).astype(jnp.int32)
    t0 = plsc.load_gather(theta_ref, [g])
    t1 = plsc.load_gather(theta_ref, [g + 1])
    return (g - 1) + (x >= t0).astype(jnp.int32) + (x >= t1).astype(jnp.int32)


def _sc_logp_kernel(pos_hbm, obs_hbm, obin_hbm, st_hbm, th1_hbm, th2_hbm,
                    ray_hbm, logp_hbm,
                    pos_v, cell_v, rays_v, obs_v, obin_v, st_v, th1_v, th2_v,
                    logp_v, sem):
    wid = lax.axis_index("s") * _NC + lax.axis_index("c")
    base = wid * _PPW
    pltpu.sync_copy(pos_hbm.at[pl.ds(base * 3, _PPW * 3)], pos_v)
    pltpu.sync_copy(obs_hbm, obs_v)
    pltpu.sync_copy(obin_hbm, obin_v)
    pltpu.sync_copy(st_hbm, st_v)
    pltpu.sync_copy(th1_hbm, th1_v)
    pltpu.sync_copy(th2_hbm, th2_v)

    iota = lax.iota(jnp.int32, 16)

    def cell_body(g, carry):
        px = (g * 16 + iota) * 3
        x = plsc.load_gather(pos_v, [px])
        y = plsc.load_gather(pos_v, [px + 1])
        ci = jnp.clip(x.astype(jnp.int32), 0, _GRID - 1)
        cj = jnp.clip(y.astype(jnp.int32), 0, _GRID - 1)
        cell_v[g // 8, pl.ds((g % 8) * 16, 16)] = ci * _GRID + cj
        return carry
    lax.fori_loop(0, _PPW // 16, cell_body, 0)

    copies = [
        pltpu.async_copy(ray_hbm.at[cell_v.at[c]],
                         rays_v.at[pl.ds(c * 128, 128)], sem)
        for c in range(_PPW // 128)
    ]
    for c in copies:
        c.wait()

    def bin_body(r, carry):
        for j in range(_ABINS // 16):
            dv = rays_v[r, pl.ds(j * 16, 16)]
            e = _bin16(dv, _C_DBIN, th2_v)
            rays_v[r, pl.ds(j * 16, 16)] = plsc.bitcast(e * _ABINS, jnp.float32)
        return carry
    lax.fori_loop(0, _PPW, bin_body, 0)

    oa_list = [obs_v[pl.ds(goff, 16)] for goff in _GSTARTS]
    ob_list = [obin_v[pl.ds(goff, 16)] for goff in _GSTARTS]
    for chunk in range(_NCHUNK):
        def p_body(p, carry):
            pg = chunk * _CHUNK + p
            th = plsc.load_gather(pos_v, [jnp.full((16,), pg * 3 + 2, jnp.int32)])
            psplat = jnp.full((16,), pg, jnp.int32)
            for gi, goff in enumerate(_GSTARTS):
                s = th + oa_list[gi]
                f = _bin16(s, _C_ABIN, th1_v)
                ai = jnp.bitwise_and(f, _ABINS - 1)
                e64 = plsc.bitcast(plsc.load_gather(rays_v, [psplat, ai]), jnp.int32)
                sidx = e64 + ob_list[gi]
                logp_v[p, pl.ds(goff, 16)] = plsc.load_gather(st_v, [sidx])
            return carry
        lax.fori_loop(0, _CHUNK, p_body, 0)
        pltpu.sync_copy(
            logp_v, logp_hbm.at[pl.ds(base + chunk * _CHUNK, _CHUNK)])


@functools.cache
def _sc_logp():
    return pl.kernel(
        _sc_logp_kernel,
        out_type=jax.ShapeDtypeStruct((_N, _B), jnp.float32),
        mesh=plsc.VectorSubcoreMesh(core_axis_name="c", subcore_axis_name="s",
                                    num_cores=_NC, num_subcores=_NS),
        compiler_params=pltpu.CompilerParams(needs_layout_passes=False,
                                             use_tc_tiling_on_sc=False),
        scratch_types=[
            pltpu.VMEM((_PPW * 3,), jnp.float32),
            pltpu.VMEM((_PPW // 128, 128), jnp.int32),
            pltpu.VMEM((_PPW, _ABINS), jnp.float32),
            pltpu.VMEM((_B + 8,), jnp.float32),
            pltpu.VMEM((_B + 8,), jnp.int32),
            pltpu.VMEM((_DBINS * _DBINS,), jnp.float32),
            pltpu.VMEM((144,), jnp.float32),
            pltpu.VMEM((80,), jnp.float32),
            pltpu.VMEM((_CHUNK, _B), jnp.float32),
            pltpu.SemaphoreType.DMA,
        ],
    )


def _sc_resample_kernel(cdf_hbm, pu_hbm, pos_hbm, out_hbm,
                        cdf_v, pu_v, pos_v, out_v):
    wid = lax.axis_index("s") * _NC + lax.axis_index("c")
    base = wid * _PPW
    pltpu.sync_copy(cdf_hbm, cdf_v)
    pltpu.sync_copy(pu_hbm.at[pl.ds(base, _PPW)], pu_v)
    pltpu.sync_copy(pos_hbm, pos_v)

    iota = lax.iota(jnp.int32, 16)

    def q_body(g, carry):
        t = pu_v[pl.ds(g * 16, 16)]
        lo = jnp.zeros((16,), jnp.int32)
        hi = jnp.full((16,), _N, jnp.int32)
        for _ in range(15):
            mid = lo + lax.shift_right_logical(hi - lo, 1)
            c = plsc.load_gather(cdf_v, [mid])
            le = t <= c
            lo = jnp.where(le, lo, mid)
            hi = jnp.where(le, mid, hi)
        idx = jnp.minimum(hi, _N - 1) * 3
        dst = (g * 16 + iota) * 3
        plsc.store_scatter(out_v, [dst], plsc.load_gather(pos_v, [idx]))
        plsc.store_scatter(out_v, [dst + 1], plsc.load_gather(pos_v, [idx + 1]))
        plsc.store_scatter(out_v, [dst + 2], plsc.load_gather(pos_v, [idx + 2]))
        return carry
    lax.fori_loop(0, _PPW // 16, q_body, 0)
    pltpu.sync_copy(out_v, out_hbm.at[pl.ds(base * 3, _PPW * 3)])


@functools.cache
def _sc_resample():
    return pl.kernel(
        _sc_resample_kernel,
        out_type=jax.ShapeDtypeStruct((_N * 3,), jnp.float32),
        mesh=plsc.VectorSubcoreMesh(core_axis_name="c", subcore_axis_name="s",
                                    num_cores=_NC, num_subcores=_NS),
        compiler_params=pltpu.CompilerParams(needs_layout_passes=False,
                                             use_tc_tiling_on_sc=False),
        scratch_types=[
            pltpu.VMEM((_N,), jnp.float32),
            pltpu.VMEM((_PPW,), jnp.float32),
            pltpu.VMEM((_N * 3,), jnp.float32),
            pltpu.VMEM((_PPW * 3,), jnp.float32),
        ],
    )


def kernel(particle_pos, log_weights, obs_dist, obs_angle, ray_map,
           sensor_table, u):
    two_pi = _TWO_PI
    obin = jnp.clip(jnp.floor(obs_dist / _MAXD * _DBINS).astype(jnp.int32),
                    0, _DBINS - 1)
    guard = jnp.where(u[0] < 2.0, jnp.float32(0.0), jnp.float32(1.0))
    th1, th2 = _build_thresholds(guard)

    pos_flat = particle_pos.reshape(-1)
    obs_pad = jnp.concatenate([obs_angle, jnp.zeros((8,), jnp.float32)])
    obin_pad = jnp.concatenate([obin, jnp.zeros((8,), jnp.int32)])
    st_flat = sensor_table.reshape(-1)

    logp = _sc_logp()(pos_flat, obs_pad, obin_pad, st_flat, th1, th2, ray_map)

    loglik = jnp.sum(logp, axis=1)
    logits = log_weights + _EVID * loglik
    logw = jax.nn.log_softmax(logits)
    w = jnp.exp(logw)
    cdf = jnp.cumsum(w)
    pos_u = (jnp.arange(_N, dtype=jnp.float32) + u[0]) / _N
    new_flat = _sc_resample()(cdf, pos_u, pos_flat)
    new_particles = new_flat.reshape(_N, 3)
    mean_pose = jnp.sum(w[:, None] * particle_pos, axis=0)
    return new_particles, logits, mean_pose


# speedup vs baseline: 278.9570x; 1.8093x over previous
import functools

import numpy as np
import jax
import jax.numpy as jnp
from jax import lax
from jax.experimental import pallas as pl
from jax.experimental.pallas import tpu as pltpu
from jax.experimental.pallas import tpu_sc as plsc

_N = 16384
_B = 360
_GRID = 256
_ABINS = 64
_DBINS = 64
_MAXD = 100.0
_EVID = 1.0
_TWO_PI = 2.0 * np.pi

_NC = 2
_NS = 16
_NW = _NC * _NS
_PPW = _N // _NW
_CHUNK = 128
_NCHUNK = _PPW // _CHUNK
_GSTARTS = tuple(list(range(0, _B - 16, 16)) + [_B - 16])

_C_ABIN = np.float32(np.float64(_ABINS) / (2.0 * np.pi))
_C_DBIN = np.float32(np.float64(_DBINS) / np.float64(_MAXD))


def _candidate_grid(centers64, width):
    c0 = centers64.astype(np.float32)
    cands = [c0]
    lo = c0.copy()
    hi = c0.copy()
    for _ in range(width):
        lo = np.nextafter(lo, np.float32(-np.inf))
        hi = np.nextafter(hi, np.float32(np.inf))
        cands.append(lo)
        cands.append(hi)
    return np.stack(cands, axis=1)


_ABIN_CANDS = _candidate_grid(np.arange(129, dtype=np.float64) * (2.0 * np.pi / 64.0), 40)
_DBIN_CANDS = _candidate_grid(np.arange(65, dtype=np.float64) * (100.0 / 64.0), 40)


def _build_thresholds(guard):
    two_pi = _TWO_PI
    s = jnp.asarray(_ABIN_CANDS) + guard
    ang = s % two_pi
    fu = jnp.clip(jnp.floor(ang / two_pi * _ABINS).astype(jnp.int32), 0, _ABINS - 1)
    fu = fu + _ABINS * (s >= np.float32(two_pi)).astype(jnp.int32)
    kk = jnp.arange(129, dtype=jnp.int32)[:, None]
    th1 = jnp.min(jnp.where(fu >= kk, s, jnp.inf), axis=1)
    th1 = th1.at[0].set(-jnp.inf)
    th1 = jnp.concatenate([th1, jnp.full((15,), jnp.inf, jnp.float32)])
    d = jnp.asarray(_DBIN_CANDS) + guard
    fu2 = jnp.clip(jnp.floor(d / _MAXD * _DBINS).astype(jnp.int32), 0, _DBINS - 1)
    kk2 = jnp.arange(65, dtype=jnp.int32)[:, None]
    th2 = jnp.min(jnp.where(fu2 >= kk2, d, jnp.inf), axis=1)
    th2 = th2.at[0].set(-jnp.inf)
    th2 = jnp.concatenate([th2, jnp.full((15,), jnp.inf, jnp.float32)])
    return th1, th2


def _bin16(x, scale, theta_ref):
    g = (x * scale).astype(jnp.int32)
    t0 = plsc.load_gather(theta_ref, [g])
    t1 = plsc.load_gather(theta_ref, [g + 1])
    return (g - 1) + (x >= t0).astype(jnp.int32) + (x >= t1).astype(jnp.int32)


def _sc_logp_kernel(pos_hbm, obs_hbm, obin_hbm, st_hbm, th1_hbm, th2_hbm,
                    ray_hbm, logp_hbm,
                    pos_v, cell_v, rays_v, obs_v, obin_v, st_v, th1_v, th2_v,
                    logp_v, sem):
    wid = lax.axis_index("s") * _NC + lax.axis_index("c")
    base = wid * _PPW
    pltpu.sync_copy(pos_hbm.at[pl.ds(base * 3, _PPW * 3)], pos_v)
    pltpu.sync_copy(obs_hbm, obs_v)
    pltpu.sync_copy(obin_hbm, obin_v)
    pltpu.sync_copy(st_hbm, st_v)
    pltpu.sync_copy(th1_hbm, th1_v)
    pltpu.sync_copy(th2_hbm, th2_v)

    iota = lax.iota(jnp.int32, 16)

    @plsc.parallel_loop(0, _PPW // 16, 1, unroll=4)
    def cell_body(g):
        px = (g * 16 + iota) * 3
        x = plsc.load_gather(pos_v, [px])
        y = plsc.load_gather(pos_v, [px + 1])
        ci = jnp.clip(x.astype(jnp.int32), 0, _GRID - 1)
        cj = jnp.clip(y.astype(jnp.int32), 0, _GRID - 1)
        cell_v[g // 8, pl.ds((g % 8) * 16, 16)] = ci * _GRID + cj

    copies = [
        pltpu.async_copy(ray_hbm.at[cell_v.at[c]],
                         rays_v.at[pl.ds(c * 128, 128)], sem)
        for c in range(_PPW // 128)
    ]
    for c in copies:
        c.wait()

    @plsc.parallel_loop(0, _PPW, 1, unroll=4)
    def bin_body(r):
        for j in range(_ABINS // 16):
            dv = rays_v[r, pl.ds(j * 16, 16)]
            e = _bin16(dv, _C_DBIN, th2_v)
            rays_v[r, pl.ds(j * 16, 16)] = plsc.bitcast(e * _ABINS, jnp.float32)

    oa_list = [obs_v[pl.ds(goff, 16)] for goff in _GSTARTS]
    ob_list = [obin_v[pl.ds(goff, 16)] for goff in _GSTARTS]
    for chunk in range(_NCHUNK):
        @plsc.parallel_loop(0, _CHUNK, 1, unroll=1)
        def p_body(p):
            pg = chunk * _CHUNK + p
            th = plsc.load_gather(pos_v, [jnp.full((16,), pg * 3 + 2, jnp.int32)])
            psplat = jnp.full((16,), pg, jnp.int32)
            for gi, goff in enumerate(_GSTARTS):
                s = th + oa_list[gi]
                f = _bin16(s, _C_ABIN, th1_v)
                ai = jnp.bitwise_and(f, _ABINS - 1)
                e64 = plsc.bitcast(plsc.load_gather(rays_v, [psplat, ai]), jnp.int32)
                sidx = e64 + ob_list[gi]
                logp_v[p, pl.ds(goff, 16)] = plsc.load_gather(st_v, [sidx])
        pltpu.sync_copy(
            logp_v, logp_hbm.at[pl.ds(base + chunk * _CHUNK, _CHUNK)])


@functools.cache
def _sc_logp():
    return pl.kernel(
        _sc_logp_kernel,
        out_type=jax.ShapeDtypeStruct((_N, _B), jnp.float32),
        mesh=plsc.VectorSubcoreMesh(core_axis_name="c", subcore_axis_name="s",
                                    num_cores=_NC, num_subcores=_NS),
        compiler_params=pltpu.CompilerParams(needs_layout_passes=False,
                                             use_tc_tiling_on_sc=False),
        scratch_types=[
            pltpu.VMEM((_PPW * 3,), jnp.float32),
            pltpu.VMEM((_PPW // 128, 128), jnp.int32),
            pltpu.VMEM((_PPW, _ABINS), jnp.float32),
            pltpu.VMEM((_B + 8,), jnp.float32),
            pltpu.VMEM((_B + 8,), jnp.int32),
            pltpu.VMEM((_DBINS * _DBINS,), jnp.float32),
            pltpu.VMEM((144,), jnp.float32),
            pltpu.VMEM((80,), jnp.float32),
            pltpu.VMEM((_CHUNK, _B), jnp.float32),
            pltpu.SemaphoreType.DMA,
        ],
    )


def _sc_resample_kernel(cdf_hbm, pu_hbm, pos_hbm, out_hbm,
                        cdf_v, pu_v, pos_v, out_v):
    wid = lax.axis_index("s") * _NC + lax.axis_index("c")
    base = wid * _PPW
    pltpu.sync_copy(cdf_hbm, cdf_v)
    pltpu.sync_copy(pu_hbm.at[pl.ds(base, _PPW)], pu_v)
    pltpu.sync_copy(pos_hbm, pos_v)

    iota = lax.iota(jnp.int32, 16)

    def q_body(g, carry):
        t = pu_v[pl.ds(g * 16, 16)]
        lo = jnp.zeros((16,), jnp.int32)
        hi = jnp.full((16,), _N, jnp.int32)
        for _ in range(15):
            mid = lo + lax.shift_right_logical(hi - lo, 1)
            c = plsc.load_gather(cdf_v, [mid])
            le = t <= c
            lo = jnp.where(le, lo, mid)
            hi = jnp.where(le, mid, hi)
        idx = jnp.minimum(hi, _N - 1) * 3
        dst = (g * 16 + iota) * 3
        plsc.store_scatter(out_v, [dst], plsc.load_gather(pos_v, [idx]))
        plsc.store_scatter(out_v, [dst + 1], plsc.load_gather(pos_v, [idx + 1]))
        plsc.store_scatter(out_v, [dst + 2], plsc.load_gather(pos_v, [idx + 2]))
        return carry
    lax.fori_loop(0, _PPW // 16, q_body, 0)
    pltpu.sync_copy(out_v, out_hbm.at[pl.ds(base * 3, _PPW * 3)])


@functools.cache
def _sc_resample():
    return pl.kernel(
        _sc_resample_kernel,
        out_type=jax.ShapeDtypeStruct((_N * 3,), jnp.float32),
        mesh=plsc.VectorSubcoreMesh(core_axis_name="c", subcore_axis_name="s",
                                    num_cores=_NC, num_subcores=_NS),
        compiler_params=pltpu.CompilerParams(needs_layout_passes=False,
                                             use_tc_tiling_on_sc=False),
        scratch_types=[
            pltpu.VMEM((_N,), jnp.float32),
            pltpu.VMEM((_PPW,), jnp.float32),
            pltpu.VMEM((_N * 3,), jnp.float32),
            pltpu.VMEM((_PPW * 3,), jnp.float32),
        ],
    )


def kernel(particle_pos, log_weights, obs_dist, obs_angle, ray_map,
           sensor_table, u):
    two_pi = _TWO_PI
    obin = jnp.clip(jnp.floor(obs_dist / _MAXD * _DBINS).astype(jnp.int32),
                    0, _DBINS - 1)
    guard = jnp.where(u[0] < 2.0, jnp.float32(0.0), jnp.float32(1.0))
    th1, th2 = _build_thresholds(guard)

    pos_flat = particle_pos.reshape(-1)
    obs_pad = jnp.concatenate([obs_angle, jnp.zeros((8,), jnp.float32)])
    obin_pad = jnp.concatenate([obin, jnp.zeros((8,), jnp.int32)])
    st_flat = sensor_table.reshape(-1)

    logp = _sc_logp()(pos_flat, obs_pad, obin_pad, st_flat, th1, th2, ray_map)

    loglik = jnp.sum(logp, axis=1)
    logits = log_weights + _EVID * loglik
    logw = jax.nn.log_softmax(logits)
    w = jnp.exp(logw)
    cdf = jnp.cumsum(w)
    pos_u = (jnp.arange(_N, dtype=jnp.float32) + u[0]) / _N
    new_flat = _sc_resample()(cdf, pos_u, pos_flat)
    new_particles = new_flat.reshape(_N, 3)
    mean_pose = jnp.sum(w[:, None] * particle_pos, axis=0)
    return new_particles, logits, mean_pose
